# baseline (speedup 1.0000x reference)
import jax
import jax.numpy as jnp
from jax import lax
from jax.experimental import pallas as pl
from jax.experimental.pallas import tpu as pltpu
from jax.experimental.pallas import tpu_sc as plsc

N = 10000
E = 320000
D = 128
G = 512

NC = 2
NS = 16
NW = NC * NS
EPS = E // NW
CHUNK = 128
KBUF = 2
NPASS = 2
WCHUNK = 40
NROUNDS = NPASS * WCHUNK // KBUF
NCHUNK = NPASS * WCHUNK
EPAD = NCHUNK * CHUNK
RSWAP = WCHUNK // KBUF
ZROWS = 632
NPAD = ZROWS * NS


def _aggr_body(h_hbm, src_hbm, dst_hbm, zeros_hbm, out_hbm,
               idx_src, idx_dst, b0, b1, s0, s1, acc):
    bufs = (b0, b1)
    ssem = (s0, s1)
    c = lax.axis_index("c")
    s = lax.axis_index("s")
    w = c * NS + s
    pltpu.async_copy(zeros_hbm.at[pl.ds(s * ZROWS, ZROWS)],
                     acc.at[pl.ds(s * ZROWS, ZROWS)], s0)
    pltpu.async_copy(src_hbm.at[w].at[0], idx_src, s1)
    pltpu.async_copy(dst_hbm.at[w].at[0], idx_dst, s1)
    pltpu.make_async_copy(zeros_hbm.at[pl.ds(s * ZROWS, ZROWS)],
                          acc.at[pl.ds(s * ZROWS, ZROWS)], s0).wait()
    pltpu.make_async_copy(src_hbm.at[w].at[0], idx_src, s1).wait()
    pltpu.make_async_copy(dst_hbm.at[w].at[0], idx_dst, s1).wait()
    plsc.subcore_barrier()

    @pl.loop(0, NROUNDS)
    def _(r):
        lj = r * KBUF - lax.select(r >= RSWAP, WCHUNK, 0)

        @pl.when(r == RSWAP)
        def _():
            for i in range(KBUF):
                pltpu.make_async_copy(
                    bufs[i], acc.at[idx_dst.at[i]], ssem[i]).wait()
            pltpu.sync_copy(src_hbm.at[w].at[1], idx_src)
            pltpu.sync_copy(dst_hbm.at[w].at[1], idx_dst)

        for i in range(KBUF):
            @pl.when(jnp.logical_and(r > 0, r != RSWAP))
            def _():
                pltpu.make_async_copy(
                    bufs[i], acc.at[idx_dst.at[lj + i]], ssem[i]).wait()

            pltpu.sync_copy(h_hbm.at[idx_src.at[lj + i]], bufs[i])
            pltpu.async_copy(bufs[i], acc.at[idx_dst.at[lj + i]], ssem[i],
                             add=True)

    for i in range(KBUF):
        pltpu.make_async_copy(
            bufs[i], acc.at[idx_dst.at[i]], ssem[i]).wait()

    plsc.subcore_barrier()
    pltpu.sync_copy(acc.at[pl.ds(s * ZROWS, ZROWS)],
                    out_hbm.at[c].at[pl.ds(s * ZROWS, ZROWS)])


@jax.jit
def _aggr_sc(h_pad, srcp, dstp, zeros):
    kern = pl.kernel(
        _aggr_body,
        out_type=jax.ShapeDtypeStruct((NC, NPAD, D), jnp.float32),
        mesh=plsc.VectorSubcoreMesh(
            core_axis_name="c", subcore_axis_name="s",
            num_cores=NC, num_subcores=NS),
        scratch_types=(
            [pltpu.VMEM((WCHUNK, CHUNK), jnp.int32),
             pltpu.VMEM((WCHUNK, CHUNK), jnp.int32)]
            + [pltpu.VMEM((CHUNK, D), jnp.float32)] * KBUF
            + [pltpu.SemaphoreType.DMA] * KBUF
            + [pltpu.VMEM_SHARED((NPAD, D), jnp.float32)]
        ),
    )
    return kern(h_pad, srcp, dstp, zeros)


ROWS_BLK = 2000
GRID = N // ROWS_BLK


def _mlp_body(x_ref, aggr_ref, w1_ref, b1_ref, w2_ref, b2_ref, batch_ref,
              h_ref, pool_ref):
    a = x_ref[...] + aggr_ref[0] + aggr_ref[1]
    z = lax.dot_general(a, w1_ref[...], (((1,), (1,)), ((), ())),
                        preferred_element_type=jnp.float32,
                        precision=lax.Precision.HIGHEST)
    z = jnp.maximum(z + b1_ref[...], 0.0)
    h = lax.dot_general(z, w2_ref[...], (((1,), (1,)), ((), ())),
                        preferred_element_type=jnp.float32,
                        precision=lax.Precision.HIGHEST)
    h = jnp.maximum(h + b2_ref[...], 0.0)
    h_ref[...] = h
    ids = batch_ref[0, 0, :]
    onehot = (ids[:, None] ==
              lax.broadcasted_iota(jnp.int32, (ROWS_BLK, G), 1)
              ).astype(jnp.float32)
    p = lax.dot_general(onehot, h, (((0,), (0,)), ((), ())),
                        preferred_element_type=jnp.float32,
                        precision=lax.Precision.HIGHEST)

    @pl.when(pl.program_id(0) == 0)
    def _():
        pool_ref[...] = jnp.zeros_like(pool_ref)

    pool_ref[...] += p


@jax.jit
def _mlp_tc(x_pad, aggr, w1f, b1f, w2, b2, batch3):
    return pl.pallas_call(
        _mlp_body,
        grid=(GRID,),
        in_specs=[
            pl.BlockSpec((ROWS_BLK, D), lambda r: (r, 0)),
            pl.BlockSpec((NC, ROWS_BLK, D), lambda r: (0, r, 0)),
            pl.BlockSpec((D, D), lambda r: (0, 0)),
            pl.BlockSpec((1, D), lambda r: (0, 0)),
            pl.BlockSpec((D, D), lambda r: (0, 0)),
            pl.BlockSpec((1, D), lambda r: (0, 0)),
            pl.BlockSpec((1, 1, ROWS_BLK), lambda r: (r, 0, 0)),
        ],
        out_specs=[
            pl.BlockSpec((ROWS_BLK, D), lambda r: (r, 0)),
            pl.BlockSpec((G, D), lambda r: (0, 0)),
        ],
        out_shape=[
            jax.ShapeDtypeStruct((NPAD, D), jnp.float32),
            jax.ShapeDtypeStruct((G, D), jnp.float32),
        ],
    )(x_pad, aggr, w1f, b1f, w2, b2, batch3)


def _mlp3_body(x_ref, aggr_ref, w1_ref, b1_ref, w2_ref, b2_ref, batch_ref,
               p1_ref, p2_ref, l1w_ref, l1b_ref, l2w_ref, l2b_ref,
               o_ref, pool_acc):
    a = x_ref[...] + aggr_ref[0] + aggr_ref[1]
    z = lax.dot_general(a, w1_ref[...], (((1,), (1,)), ((), ())),
                        preferred_element_type=jnp.float32,
                        precision=lax.Precision.HIGHEST)
    z = jnp.maximum(z + b1_ref[...], 0.0)
    h = lax.dot_general(z, w2_ref[...], (((1,), (1,)), ((), ())),
                        preferred_element_type=jnp.float32,
                        precision=lax.Precision.HIGHEST)
    h = jnp.maximum(h + b2_ref[...], 0.0)
    ids = batch_ref[0, 0, :]
    onehot = (ids[:, None] ==
              lax.broadcasted_iota(jnp.int32, (ROWS_BLK, G), 1)
              ).astype(jnp.float32)
    p = lax.dot_general(onehot, h, (((0,), (0,)), ((), ())),
                        preferred_element_type=jnp.float32,
                        precision=lax.Precision.HIGHEST)

    @pl.when(pl.program_id(0) == 0)
    def _():
        pool_acc[...] = jnp.zeros_like(pool_acc)

    pool_acc[...] += p

    @pl.when(pl.program_id(0) == GRID - 1)
    def _():
        hc = jnp.concatenate([p1_ref[...], p2_ref[...], pool_acc[...]],
                             axis=1)
        zc = lax.dot_general(hc, l1w_ref[...], (((1,), (1,)), ((), ())),
                             preferred_element_type=jnp.float32,
                             precision=lax.Precision.HIGHEST)
        zc = jnp.maximum(zc + l1b_ref[...], 0.0)
        oc = lax.dot_general(zc, l2w_ref[...], (((1,), (1,)), ((), ())),
                             preferred_element_type=jnp.float32,
                             precision=lax.Precision.HIGHEST)
        o_ref[...] = oc + l2b_ref[...]


@jax.jit
def _mlp3_tc(x_pad, aggr, w1f, b1f, w2, b2, batch3, p1, p2,
             l1w, l1b, l2w, l2b):
    L = 3 * D
    return pl.pallas_call(
        _mlp3_body,
        grid=(GRID,),
        in_specs=[
            pl.BlockSpec((ROWS_BLK, D), lambda r: (r, 0)),
            pl.BlockSpec((NC, ROWS_BLK, D), lambda r: (0, r, 0)),
            pl.BlockSpec((D, D), lambda r: (0, 0)),
            pl.BlockSpec((1, D), lambda r: (0, 0)),
            pl.BlockSpec((D, D), lambda r: (0, 0)),
            pl.BlockSpec((1, D), lambda r: (0, 0)),
            pl.BlockSpec((1, 1, ROWS_BLK), lambda r: (r, 0, 0)),
            pl.BlockSpec((G, D), lambda r: (0, 0)),
            pl.BlockSpec((G, D), lambda r: (0, 0)),
            pl.BlockSpec((L, L), lambda r: (0, 0)),
            pl.BlockSpec((1, L), lambda r: (0, 0)),
            pl.BlockSpec((D, L), lambda r: (0, 0)),
            pl.BlockSpec((1, D), lambda r: (0, 0)),
        ],
        out_specs=pl.BlockSpec((G, D), lambda r: (0, 0)),
        out_shape=jax.ShapeDtypeStruct((G, D), jnp.float32),
        scratch_shapes=[pltpu.VMEM((G, D), jnp.float32)],
    )(x_pad, aggr, w1f, b1f, w2, b2, batch3, p1, p2, l1w, l1b, l2w, l2b)


def _fold_bn(W1, b1, g, be):
    s = g / jnp.sqrt(1.0 + 1e-5)
    return W1 * s[:, None], (b1 * s + be)[None, :]


def kernel(x, edge_index, batch, c1_W1, c1_b1, c1_g, c1_be, c1_W2, c1_b2,
           c2_W1, c2_b1, c2_g, c2_be, c2_W2, c2_b2,
           c3_W1, c3_b1, c3_g, c3_be, c3_W2, c3_b2,
           lin1_W, lin1_b, lin2_W, lin2_b):
    src, dst = edge_index[0], edge_index[1]
    pad_src = (jnp.arange(NW * (EPAD - EPS), dtype=jnp.int32) * 131 + 7) % N
    srcp = jnp.concatenate(
        [src.reshape(NW, EPS), pad_src.reshape(NW, EPAD - EPS)], axis=1
    ).reshape(NW, NPASS, WCHUNK, CHUNK)
    dstp = jnp.pad(dst.reshape(NW, EPS), ((0, 0), (0, EPAD - EPS)),
                   constant_values=N).reshape(NW, NPASS, WCHUNK, CHUNK)
    zeros = jnp.zeros((NPAD, D), jnp.float32)
    batch3 = batch.reshape(GRID, 1, ROWS_BLK)

    x_pad = jnp.pad(x, ((0, NPAD - N), (0, 0)))
    w1f1, b1f1 = _fold_bn(c1_W1, c1_b1, c1_g, c1_be)
    w1f2, b1f2 = _fold_bn(c2_W1, c2_b1, c2_g, c2_be)
    w1f3, b1f3 = _fold_bn(c3_W1, c3_b1, c3_g, c3_be)

    aggr1 = _aggr_sc(x_pad, srcp, dstp, zeros)
    h1, p1 = _mlp_tc(x_pad, aggr1, w1f1, b1f1, c1_W2, c1_b2[None, :], batch3)
    aggr2 = _aggr_sc(h1, srcp, dstp, zeros)
    h2, p2 = _mlp_tc(h1, aggr2, w1f2, b1f2, c2_W2, c2_b2[None, :], batch3)
    aggr3 = _aggr_sc(h2, srcp, dstp, zeros)
    return _mlp3_tc(h2, aggr3, w1f3, b1f3, c3_W2, c3_b2[None, :], batch3,
                    p1, p2, lin1_W, lin1_b[None, :],
                    lin2_W, lin2_b[None, :])


# speedup vs baseline: 1.1783x; 1.1783x over previous
import jax
import jax.numpy as jnp
from jax import lax
from jax.experimental import pallas as pl
from jax.experimental.pallas import tpu as pltpu
from jax.experimental.pallas import tpu_sc as plsc

N = 10000
E = 320000
D = 128
G = 512

NC = 2
NS = 16
NW = NC * NS
EPS = E // NW
CHUNK = 128
KBUF = 2
NPASS = 2
WCHUNK = 40
NROUNDS = NPASS * WCHUNK // KBUF
NCHUNK = NPASS * WCHUNK
EPAD = NCHUNK * CHUNK
RSWAP = WCHUNK // KBUF
ZROWS = 632
NPAD = ZROWS * NS


def _aggr_body(h_hbm, src_hbm, dst_hbm, zeros_hbm, out_hbm,
               idx_src, idx_dst, b0, b1, s0, s1, acc):
    bufs = (b0, b1)
    ssem = (s0, s1)
    c = lax.axis_index("c")
    s = lax.axis_index("s")
    w = c * NS + s
    pltpu.sync_copy(zeros_hbm.at[pl.ds(s * ZROWS, ZROWS)],
                    acc.at[pl.ds(s * ZROWS, ZROWS)])
    pltpu.sync_copy(src_hbm.at[w].at[0], idx_src)
    pltpu.sync_copy(dst_hbm.at[w].at[0], idx_dst)
    plsc.subcore_barrier()

    @pl.loop(0, NROUNDS)
    def _(r):
        lj = r * KBUF - lax.select(r >= RSWAP, WCHUNK, 0)

        @pl.when(r == RSWAP)
        def _():
            for i in range(KBUF):
                pltpu.make_async_copy(
                    bufs[i], acc.at[idx_dst.at[i]], ssem[i]).wait()
            pltpu.sync_copy(src_hbm.at[w].at[1], idx_src)
            pltpu.sync_copy(dst_hbm.at[w].at[1], idx_dst)

        for i in range(KBUF):
            @pl.when(jnp.logical_and(r > 0, r != RSWAP))
            def _():
                pltpu.make_async_copy(
                    bufs[i], acc.at[idx_dst.at[lj + i]], ssem[i]).wait()

            pltpu.sync_copy(h_hbm.at[idx_src.at[lj + i]], bufs[i])
            pltpu.async_copy(bufs[i], acc.at[idx_dst.at[lj + i]], ssem[i],
                             add=True)

    for i in range(KBUF):
        pltpu.make_async_copy(
            bufs[i], acc.at[idx_dst.at[i]], ssem[i]).wait()

    plsc.subcore_barrier()
    pltpu.sync_copy(acc.at[pl.ds(s * ZROWS, ZROWS)],
                    out_hbm.at[c].at[pl.ds(s * ZROWS, ZROWS)])


@jax.jit
def _aggr_sc(h_pad, srcp, dstp, zeros):
    kern = pl.kernel(
        _aggr_body,
        out_type=jax.ShapeDtypeStruct((NC, NPAD, D), jnp.float32),
        mesh=plsc.VectorSubcoreMesh(
            core_axis_name="c", subcore_axis_name="s",
            num_cores=NC, num_subcores=NS),
        scratch_types=(
            [pltpu.VMEM((WCHUNK, CHUNK), jnp.int32),
             pltpu.VMEM((WCHUNK, CHUNK), jnp.int32)]
            + [pltpu.VMEM((CHUNK, D), jnp.float32)] * KBUF
            + [pltpu.SemaphoreType.DMA] * KBUF
            + [pltpu.VMEM_SHARED((NPAD, D), jnp.float32)]
        ),
    )
    return kern(h_pad, srcp, dstp, zeros)


ROWS_BLK = 2000
GRID = N // ROWS_BLK


def _mlp_body(x_ref, aggr_ref, w1_ref, b1_ref, w2_ref, b2_ref, batch_ref,
              h_ref, pool_ref):
    a = x_ref[...] + aggr_ref[0] + aggr_ref[1]
    z = lax.dot_general(a, w1_ref[...], (((1,), (1,)), ((), ())),
                        preferred_element_type=jnp.float32,
                        precision=lax.Precision.HIGHEST)
    z = jnp.maximum(z + b1_ref[...], 0.0)
    h = lax.dot_general(z, w2_ref[...], (((1,), (1,)), ((), ())),
                        preferred_element_type=jnp.float32,
                        precision=lax.Precision.HIGHEST)
    h = jnp.maximum(h + b2_ref[...], 0.0)
    h_ref[...] = h
    ids = batch_ref[0, 0, :]
    onehot = (ids[:, None] ==
              lax.broadcasted_iota(jnp.int32, (ROWS_BLK, G), 1)
              ).astype(jnp.float32)
    p = lax.dot_general(onehot, h, (((0,), (0,)), ((), ())),
                        preferred_element_type=jnp.float32,
                        precision=lax.Precision.HIGHEST)

    @pl.when(pl.program_id(0) == 0)
    def _():
        pool_ref[...] = jnp.zeros_like(pool_ref)

    pool_ref[...] += p


@jax.jit
def _mlp_tc(x_pad, aggr, w1f, b1f, w2, b2, batch3):
    return pl.pallas_call(
        _mlp_body,
        grid=(GRID,),
        in_specs=[
            pl.BlockSpec((ROWS_BLK, D), lambda r: (r, 0)),
            pl.BlockSpec((NC, ROWS_BLK, D), lambda r: (0, r, 0)),
            pl.BlockSpec((D, D), lambda r: (0, 0)),
            pl.BlockSpec((1, D), lambda r: (0, 0)),
            pl.BlockSpec((D, D), lambda r: (0, 0)),
            pl.BlockSpec((1, D), lambda r: (0, 0)),
            pl.BlockSpec((1, 1, ROWS_BLK), lambda r: (r, 0, 0)),
        ],
        out_specs=[
            pl.BlockSpec((ROWS_BLK, D), lambda r: (r, 0)),
            pl.BlockSpec((G, D), lambda r: (0, 0)),
        ],
        out_shape=[
            jax.ShapeDtypeStruct((NPAD, D), jnp.float32),
            jax.ShapeDtypeStruct((G, D), jnp.float32),
        ],
    )(x_pad, aggr, w1f, b1f, w2, b2, batch3)


def _mlp3_body(x_ref, aggr_ref, w1_ref, b1_ref, w2_ref, b2_ref, batch_ref,
               p1_ref, p2_ref, l1w_ref, l1b_ref, l2w_ref, l2b_ref,
               o_ref, pool_acc):
    a = x_ref[...] + aggr_ref[0] + aggr_ref[1]
    z = lax.dot_general(a, w1_ref[...], (((1,), (1,)), ((), ())),
                        preferred_element_type=jnp.float32,
                        precision=lax.Precision.HIGHEST)
    z = jnp.maximum(z + b1_ref[...], 0.0)
    h = lax.dot_general(z, w2_ref[...], (((1,), (1,)), ((), ())),
                        preferred_element_type=jnp.float32,
                        precision=lax.Precision.HIGHEST)
    h = jnp.maximum(h + b2_ref[...], 0.0)
    ids = batch_ref[0, 0, :]
    onehot = (ids[:, None] ==
              lax.broadcasted_iota(jnp.int32, (ROWS_BLK, G), 1)
              ).astype(jnp.float32)
    p = lax.dot_general(onehot, h, (((0,), (0,)), ((), ())),
                        preferred_element_type=jnp.float32,
                        precision=lax.Precision.HIGHEST)

    @pl.when(pl.program_id(0) == 0)
    def _():
        pool_acc[...] = jnp.zeros_like(pool_acc)

    pool_acc[...] += p

    @pl.when(pl.program_id(0) == GRID - 1)
    def _():
        hc = jnp.concatenate([p1_ref[...], p2_ref[...], pool_acc[...]],
                             axis=1)
        zc = lax.dot_general(hc, l1w_ref[...], (((1,), (1,)), ((), ())),
                             preferred_element_type=jnp.float32,
                             precision=lax.Precision.HIGHEST)
        zc = jnp.maximum(zc + l1b_ref[...], 0.0)
        oc = lax.dot_general(zc, l2w_ref[...], (((1,), (1,)), ((), ())),
                             preferred_element_type=jnp.float32,
                             precision=lax.Precision.HIGHEST)
        o_ref[...] = oc + l2b_ref[...]


@jax.jit
def _mlp3_tc(x_pad, aggr, w1f, b1f, w2, b2, batch3, p1, p2,
             l1w, l1b, l2w, l2b):
    L = 3 * D
    return pl.pallas_call(
        _mlp3_body,
        grid=(GRID,),
        in_specs=[
            pl.BlockSpec((ROWS_BLK, D), lambda r: (r, 0)),
            pl.BlockSpec((NC, ROWS_BLK, D), lambda r: (0, r, 0)),
            pl.BlockSpec((D, D), lambda r: (0, 0)),
            pl.BlockSpec((1, D), lambda r: (0, 0)),
            pl.BlockSpec((D, D), lambda r: (0, 0)),
            pl.BlockSpec((1, D), lambda r: (0, 0)),
            pl.BlockSpec((1, 1, ROWS_BLK), lambda r: (r, 0, 0)),
            pl.BlockSpec((G, D), lambda r: (0, 0)),
            pl.BlockSpec((G, D), lambda r: (0, 0)),
            pl.BlockSpec((L, L), lambda r: (0, 0)),
            pl.BlockSpec((1, L), lambda r: (0, 0)),
            pl.BlockSpec((D, L), lambda r: (0, 0)),
            pl.BlockSpec((1, D), lambda r: (0, 0)),
        ],
        out_specs=pl.BlockSpec((G, D), lambda r: (0, 0)),
        out_shape=jax.ShapeDtypeStruct((G, D), jnp.float32),
        scratch_shapes=[pltpu.VMEM((G, D), jnp.float32)],
    )(x_pad, aggr, w1f, b1f, w2, b2, batch3, p1, p2, l1w, l1b, l2w, l2b)


def _fold_bn(W1, b1, g, be):
    s = g / jnp.sqrt(1.0 + 1e-5)
    return W1 * s[:, None], (b1 * s + be)[None, :]


def kernel(x, edge_index, batch, c1_W1, c1_b1, c1_g, c1_be, c1_W2, c1_b2,
           c2_W1, c2_b1, c2_g, c2_be, c2_W2, c2_b2,
           c3_W1, c3_b1, c3_g, c3_be, c3_W2, c3_b2,
           lin1_W, lin1_b, lin2_W, lin2_b):
    src, dst = edge_index[0], edge_index[1]
    pad_src = (jnp.arange(NW * (EPAD - EPS), dtype=jnp.int32) * 131 + 7) % N
    srcp = jnp.concatenate(
        [src.reshape(NW, EPS), pad_src.reshape(NW, EPAD - EPS)], axis=1
    ).reshape(NW, NPASS, WCHUNK, CHUNK)
    dstp = jnp.pad(dst.reshape(NW, EPS), ((0, 0), (0, EPAD - EPS)),
                   constant_values=N).reshape(NW, NPASS, WCHUNK, CHUNK)
    zeros = jnp.zeros((NPAD, D), jnp.float32)
    batch3 = batch.reshape(GRID, 1, ROWS_BLK)

    x_pad = jnp.pad(x, ((0, NPAD - N), (0, 0)))
    w1f1, b1f1 = _fold_bn(c1_W1, c1_b1, c1_g, c1_be)
    w1f2, b1f2 = _fold_bn(c2_W1, c2_b1, c2_g, c2_be)
    w1f3, b1f3 = _fold_bn(c3_W1, c3_b1, c3_g, c3_be)

    aggr1 = _aggr_sc(x_pad, srcp, dstp, zeros)
    h1, p1 = _mlp_tc(x_pad, aggr1, w1f1, b1f1, c1_W2, c1_b2[None, :], batch3)
    aggr2 = _aggr_sc(h1, srcp, dstp, zeros)
    h2, p2 = _mlp_tc(h1, aggr2, w1f2, b1f2, c2_W2, c2_b2[None, :], batch3)
    aggr3 = _aggr_sc(h2, srcp, dstp, zeros)
    return _mlp3_tc(h2, aggr3, w1f3, b1f3, c3_W2, c3_b2[None, :], batch3,
                    p1, p2, lin1_W, lin1_b[None, :],
                    lin2_W, lin2_b[None, :])
